# baseline (speedup 1.0000x reference)
import functools

import jax
import jax.numpy as jnp
from jax.experimental import pallas as pl
from jax.experimental.pallas import tpu as pltpu


def _net_body(x_ref, acat_ref, wfc_ref, bfc_ref, out_ref, xs_ref, *,
              b_blk, h_in, w_in, sb, n_chunks, chunk, n_ch, n_cls):
    h_out = h_in - 2
    w_pad = xs_ref.shape[-1]

    for b in range(b_blk):
        xw = x_ref[b, 0]
        xs_ref[b, 0:h_in, 0:w_in] = xw
        xs_ref[b, h_in:h_in + 1, :] = jnp.ones((1, w_pad), jnp.float32)
        xs_ref[b, h_in + 1:sb, :] = jnp.zeros((sb - h_in - 1, w_pad),
                                              jnp.float32)
        xs_ref[b, sb:sb + h_in, 0:w_in - 1] = xw[:, 1:]
        xs_ref[b, sb + h_in:2 * sb, :] = jnp.zeros((sb - h_in, w_pad),
                                                   jnp.float32)
        xs_ref[b, 2 * sb:2 * sb + h_in, 0:w_in - 2] = xw[:, 2:]
        xs_ref[b, 2 * sb + h_in:3 * sb, :] = jnp.zeros((sb - h_in, w_pad),
                                                       jnp.float32)

    acatt = acat_ref[...]
    logits = [None] * b_blk
    for ci in range(n_chunks):
        c0 = ci * chunk
        wc = wfc_ref[:, c0:c0 + chunk]
        for b in range(b_blk):
            convt = jax.lax.dot_general(
                xs_ref[b, :, c0:c0 + chunk], acatt,
                dimension_numbers=(((0,), (0,)), ((), ())),
                preferred_element_type=jnp.float32)
            actt = jnp.maximum(convt, 0.0)
            part = jnp.dot(wc, actt, preferred_element_type=jnp.float32)
            logits[b] = part if logits[b] is None else logits[b] + part

    bfc = bfc_ref[...]
    for b in range(b_blk):
        z = jnp.swapaxes(logits[b], 0, 1) + bfc
        z = z.reshape(n_ch, h_out, n_cls)
        m = jnp.max(z, axis=0, keepdims=True)
        s = z - m
        lse = jnp.log(jnp.sum(jnp.exp(s), axis=0, keepdims=True))
        out_ref[b] = s - lse


def _build_acat(conv_w, conv_b, h_out, sb):
    n_ch = conv_w.shape[0]
    h_in = h_out + 2
    w = conv_w.reshape(n_ch, 3, 3).astype(jnp.float32)
    hh = jnp.arange(h_out)
    tt = jnp.arange(h_in)
    kk = jnp.arange(3)
    sel = (tt[None, :, None] == hh[:, None, None] + kk[None, None, :])
    sel = sel.astype(jnp.float32)
    cols = []
    for kj in range(3):
        blk = jnp.einsum('htk,ck->cht', sel, w[:, :, kj])
        blk = blk.reshape(h_out * n_ch, h_in)
        pad = jnp.zeros((h_out * n_ch, sb - h_in), jnp.float32)
        if kj == 0:
            bias_col = jnp.repeat(conv_b.astype(jnp.float32),
                                  h_out).reshape(h_out * n_ch, 1)
            pad = jnp.concatenate(
                [bias_col,
                 jnp.zeros((h_out * n_ch, sb - h_in - 1), jnp.float32)],
                axis=1)
        cols.append(jnp.concatenate([blk, pad], axis=1))
    return jnp.concatenate(cols, axis=1)


def kernel(x, conv_w, conv_b, fc_w, fc_b):
    n, c_in, h_in, w_in = x.shape
    assert c_in == 1
    h_out = h_in - 2
    w_out = w_in - 2
    n_ch = conv_w.shape[0]
    n_cls = fc_w.shape[0]
    chunk = 256
    assert w_out % chunk == 0
    n_chunks = w_out // chunk
    b_blk = 2 if n % 2 == 0 else 1
    sb = ((h_in + 1 + 7) // 8) * 8
    n_rows = n_ch * h_out
    w_pad = ((w_in + 127) // 128) * 128

    acatt = _build_acat(conv_w, conv_b, h_out, sb).T
    bfc2 = fc_b.astype(jnp.float32).reshape(1, n_cls)

    body = functools.partial(
        _net_body, b_blk=b_blk, h_in=h_in, w_in=w_in, sb=sb,
        n_chunks=n_chunks, chunk=chunk, n_ch=n_ch, n_cls=n_cls)

    out = pl.pallas_call(
        body,
        out_shape=jax.ShapeDtypeStruct((n, n_ch, h_out, n_cls), jnp.float32),
        grid=(n // b_blk,),
        in_specs=[
            pl.BlockSpec((b_blk, 1, h_in, w_in), lambda i: (i, 0, 0, 0)),
            pl.BlockSpec((3 * sb, n_rows), lambda i: (0, 0)),
            pl.BlockSpec((n_cls, w_out), lambda i: (0, 0)),
            pl.BlockSpec((1, n_cls), lambda i: (0, 0)),
        ],
        out_specs=pl.BlockSpec((b_blk, n_ch, h_out, n_cls),
                               lambda i: (i, 0, 0, 0)),
        scratch_shapes=[pltpu.VMEM((b_blk, 3 * sb, w_pad), jnp.float32)],
        compiler_params=pltpu.CompilerParams(
            dimension_semantics=("parallel",),
            vmem_limit_bytes=64 * 1024 * 1024),
    )(x, acatt, fc_w.astype(jnp.float32), bfc2)

    return out


# speedup vs baseline: 1.7996x; 1.7996x over previous
import functools

import jax
import jax.numpy as jnp
from jax.experimental import pallas as pl
from jax.experimental.pallas import tpu as pltpu


def _net_body(x_ref, acat_ref, wfct_ref, bfc_ref, out_ref, xs_ref, *,
              b_blk, h_in, w_in, sb, n_chunks, chunk, n_ch, n_cls):
    h_out = h_in - 2
    w_pad = xs_ref.shape[-1]

    for b in range(b_blk):
        xw = x_ref[b, 0]
        xs_ref[b, 0:h_in, 0:w_in] = xw
        xs_ref[b, h_in:h_in + 1, :] = jnp.ones((1, w_pad), jnp.float32)
        xs_ref[b, h_in + 1:sb, :] = jnp.zeros((sb - h_in - 1, w_pad),
                                              jnp.float32)
        xs_ref[b, sb:sb + h_in, 0:w_in - 1] = xw[:, 1:]
        xs_ref[b, sb + h_in:2 * sb, :] = jnp.zeros((sb - h_in, w_pad),
                                                   jnp.float32)
        xs_ref[b, 2 * sb:2 * sb + h_in, 0:w_in - 2] = xw[:, 2:]
        xs_ref[b, 2 * sb + h_in:3 * sb, :] = jnp.zeros((sb - h_in, w_pad),
                                                       jnp.float32)

    acat = acat_ref[...]
    logits = [None] * b_blk
    for ci in range(n_chunks):
        c0 = ci * chunk
        wc = wfct_ref[:, c0:c0 + chunk]
        for b in range(b_blk):
            conv = jnp.dot(acat, xs_ref[b, :, c0:c0 + chunk],
                           preferred_element_type=jnp.float32)
            act = jnp.maximum(conv, 0.0)
            part = jax.lax.dot_general(
                wc, act, dimension_numbers=(((1,), (1,)), ((), ())),
                preferred_element_type=jnp.float32)
            logits[b] = part if logits[b] is None else logits[b] + part

    bfc = bfc_ref[...]
    for b in range(b_blk):
        z = jnp.swapaxes(logits[b], 0, 1) + bfc
        z = z.reshape(n_ch, h_out, n_cls)
        m = jnp.max(z, axis=0, keepdims=True)
        s = z - m
        lse = jnp.log(jnp.sum(jnp.exp(s), axis=0, keepdims=True))
        out_ref[b] = s - lse


def _build_acat(conv_w, conv_b, h_out, sb):
    n_ch = conv_w.shape[0]
    h_in = h_out + 2
    w = conv_w.reshape(n_ch, 3, 3).astype(jnp.float32)
    hh = jnp.arange(h_out)
    tt = jnp.arange(h_in)
    kk = jnp.arange(3)
    sel = (tt[None, :, None] == hh[:, None, None] + kk[None, None, :])
    sel = sel.astype(jnp.float32)
    cols = []
    for kj in range(3):
        blk = jnp.einsum('htk,ck->cht', sel, w[:, :, kj])
        blk = blk.reshape(h_out * n_ch, h_in)
        pad = jnp.zeros((h_out * n_ch, sb - h_in), jnp.float32)
        if kj == 0:
            bias_col = jnp.repeat(conv_b.astype(jnp.float32),
                                  h_out).reshape(h_out * n_ch, 1)
            pad = jnp.concatenate(
                [bias_col,
                 jnp.zeros((h_out * n_ch, sb - h_in - 1), jnp.float32)],
                axis=1)
        cols.append(jnp.concatenate([blk, pad], axis=1))
    return jnp.concatenate(cols, axis=1)


def kernel(x, conv_w, conv_b, fc_w, fc_b):
    n, c_in, h_in, w_in = x.shape
    assert c_in == 1
    h_out = h_in - 2
    w_out = w_in - 2
    n_ch = conv_w.shape[0]
    n_cls = fc_w.shape[0]
    chunk = 256
    assert w_out % chunk == 0
    n_chunks = w_out // chunk
    b_blk = 2 if n % 2 == 0 else 1
    sb = ((h_in + 1 + 7) // 8) * 8
    n_rows = n_ch * h_out
    w_pad = ((w_in + 127) // 128) * 128

    acat = _build_acat(conv_w, conv_b, h_out, sb)
    bfc2 = fc_b.astype(jnp.float32).reshape(1, n_cls)

    body = functools.partial(
        _net_body, b_blk=b_blk, h_in=h_in, w_in=w_in, sb=sb,
        n_chunks=n_chunks, chunk=chunk, n_ch=n_ch, n_cls=n_cls)

    out = pl.pallas_call(
        body,
        out_shape=jax.ShapeDtypeStruct((n, n_ch, h_out, n_cls), jnp.float32),
        grid=(n // b_blk,),
        in_specs=[
            pl.BlockSpec((b_blk, 1, h_in, w_in), lambda i: (i, 0, 0, 0)),
            pl.BlockSpec((n_rows, 3 * sb), lambda i: (0, 0)),
            pl.BlockSpec((n_cls, w_out), lambda i: (0, 0)),
            pl.BlockSpec((1, n_cls), lambda i: (0, 0)),
        ],
        out_specs=pl.BlockSpec((b_blk, n_ch, h_out, n_cls),
                               lambda i: (i, 0, 0, 0)),
        scratch_shapes=[pltpu.VMEM((b_blk, 3 * sb, w_pad), jnp.float32)],
        compiler_params=pltpu.CompilerParams(
            dimension_semantics=("parallel",),
            vmem_limit_bytes=64 * 1024 * 1024),
    )(x, acat, fc_w.astype(jnp.float32), bfc2)

    return out


# speedup vs baseline: 2.3029x; 1.2797x over previous
import functools

import jax
import jax.numpy as jnp
from jax.experimental import pallas as pl
from jax.experimental.pallas import tpu as pltpu


def _net_body(x_ref, acat_ref, wfct_ref, bfc_ref, out_ref, xs_ref, *,
              b_blk, h_in, w_in, sb, n_chunks, chunk, n_ch, n_cls):
    h_out = h_in - 2
    w_pad = xs_ref.shape[-1]

    for b in range(b_blk):
        xw = x_ref[b, 0]
        xs_ref[b, 0:h_in, 0:w_in] = xw
        xs_ref[b, h_in:h_in + 1, :] = jnp.ones((1, w_pad), jnp.float32)
        xs_ref[b, h_in + 1:sb, :] = jnp.zeros((sb - h_in - 1, w_pad),
                                              jnp.float32)
        xs_ref[b, sb:sb + h_in, 0:w_in - 1] = xw[:, 1:]
        xs_ref[b, sb + h_in:2 * sb, :] = jnp.zeros((sb - h_in, w_pad),
                                                   jnp.float32)
        xs_ref[b, 2 * sb:2 * sb + h_in, 0:w_in - 2] = xw[:, 2:]
        xs_ref[b, 2 * sb + h_in:3 * sb, :] = jnp.zeros((sb - h_in, w_pad),
                                                       jnp.float32)

    acat = acat_ref[...]
    logits = [None] * b_blk
    for ci in range(n_chunks):
        c0 = ci * chunk
        wc = wfct_ref[c0:c0 + chunk, :]
        for b in range(b_blk):
            conv = jnp.dot(acat, xs_ref[b, :, c0:c0 + chunk],
                           preferred_element_type=jnp.float32)
            act = jnp.maximum(conv, 0.0)
            part = jnp.dot(act, wc, preferred_element_type=jnp.float32)
            logits[b] = part if logits[b] is None else logits[b] + part

    bfc = bfc_ref[...]
    for b in range(b_blk):
        z = (logits[b] + bfc).reshape(n_ch, h_out, n_cls)
        m = jnp.max(z, axis=0, keepdims=True)
        s = z - m
        lse = jnp.log(jnp.sum(jnp.exp(s), axis=0, keepdims=True))
        out_ref[b] = s - lse


def _build_acat(conv_w, conv_b, h_out, sb):
    n_ch = conv_w.shape[0]
    h_in = h_out + 2
    w = conv_w.reshape(n_ch, 3, 3).astype(jnp.float32)
    hh = jnp.arange(h_out)
    tt = jnp.arange(h_in)
    kk = jnp.arange(3)
    sel = (tt[None, :, None] == hh[:, None, None] + kk[None, None, :])
    sel = sel.astype(jnp.float32)
    cols = []
    for kj in range(3):
        blk = jnp.einsum('htk,ck->cht', sel, w[:, :, kj])
        blk = blk.reshape(h_out * n_ch, h_in)
        pad = jnp.zeros((h_out * n_ch, sb - h_in), jnp.float32)
        if kj == 0:
            bias_col = jnp.repeat(conv_b.astype(jnp.float32),
                                  h_out).reshape(h_out * n_ch, 1)
            pad = jnp.concatenate(
                [bias_col,
                 jnp.zeros((h_out * n_ch, sb - h_in - 1), jnp.float32)],
                axis=1)
        cols.append(jnp.concatenate([blk, pad], axis=1))
    return jnp.concatenate(cols, axis=1)


def kernel(x, conv_w, conv_b, fc_w, fc_b):
    n, c_in, h_in, w_in = x.shape
    assert c_in == 1
    h_out = h_in - 2
    w_out = w_in - 2
    n_ch = conv_w.shape[0]
    n_cls = fc_w.shape[0]
    chunk = 256
    assert w_out % chunk == 0
    n_chunks = w_out // chunk
    b_blk = 4 if n % 4 == 0 else (2 if n % 2 == 0 else 1)
    sb = ((h_in + 1 + 7) // 8) * 8
    n_rows = n_ch * h_out
    w_pad = ((w_in + 127) // 128) * 128

    acat = _build_acat(conv_w, conv_b, h_out, sb)
    wfct = fc_w.astype(jnp.float32).T
    bfc2 = fc_b.astype(jnp.float32).reshape(1, n_cls)

    body = functools.partial(
        _net_body, b_blk=b_blk, h_in=h_in, w_in=w_in, sb=sb,
        n_chunks=n_chunks, chunk=chunk, n_ch=n_ch, n_cls=n_cls)

    out = pl.pallas_call(
        body,
        out_shape=jax.ShapeDtypeStruct((n, n_ch, h_out, n_cls), jnp.float32),
        grid=(n // b_blk,),
        in_specs=[
            pl.BlockSpec((b_blk, 1, h_in, w_in), lambda i: (i, 0, 0, 0)),
            pl.BlockSpec((n_rows, 3 * sb), lambda i: (0, 0)),
            pl.BlockSpec((w_out, n_cls), lambda i: (0, 0)),
            pl.BlockSpec((1, n_cls), lambda i: (0, 0)),
        ],
        out_specs=pl.BlockSpec((b_blk, n_ch, h_out, n_cls),
                               lambda i: (i, 0, 0, 0)),
        scratch_shapes=[pltpu.VMEM((b_blk, 3 * sb, w_pad), jnp.float32)],
        compiler_params=pltpu.CompilerParams(
            dimension_semantics=("parallel",),
            vmem_limit_bytes=64 * 1024 * 1024),
    )(x, acat, wfct, bfc2)

    return out


# speedup vs baseline: 2.3326x; 1.0129x over previous
import functools

import jax
import jax.numpy as jnp
from jax.experimental import pallas as pl
from jax.experimental.pallas import tpu as pltpu


def _net_body(x_ref, acat_ref, wfct_ref, bfc_ref, out_ref, xs_ref, *,
              b_blk, h_in, w_in, sb, n_chunks, chunk, n_ch, n_cls):
    h_out = h_in - 2
    w_pad = xs_ref.shape[-1]

    for b in range(b_blk):
        xw = x_ref[b, 0]
        xs_ref[b, 0:h_in, 0:w_in] = xw
        xs_ref[b, h_in:h_in + 1, :] = jnp.ones((1, w_pad), jnp.float32)
        xs_ref[b, h_in + 1:sb, :] = jnp.zeros((sb - h_in - 1, w_pad),
                                              jnp.float32)
        xs_ref[b, sb:sb + h_in, 0:w_in - 1] = xw[:, 1:]
        xs_ref[b, sb + h_in:2 * sb, :] = jnp.zeros((sb - h_in, w_pad),
                                                   jnp.float32)
        xs_ref[b, 2 * sb:2 * sb + h_in, 0:w_in - 2] = xw[:, 2:]
        xs_ref[b, 2 * sb + h_in:3 * sb, :] = jnp.zeros((sb - h_in, w_pad),
                                                       jnp.float32)

    acat = acat_ref[...]
    logits = [None] * b_blk
    for ci in range(n_chunks):
        c0 = ci * chunk
        wc = wfct_ref[c0:c0 + chunk, :]
        for b in range(b_blk):
            conv = jnp.dot(acat, xs_ref[b, :, c0:c0 + chunk],
                           preferred_element_type=jnp.float32)
            act = jnp.maximum(conv, 0.0)
            part = jnp.dot(act, wc, preferred_element_type=jnp.float32)
            logits[b] = part if logits[b] is None else logits[b] + part

    bfc = bfc_ref[...]
    for b in range(b_blk):
        z = (logits[b] + bfc).reshape(n_ch, h_out, n_cls)
        m = jnp.max(z, axis=0, keepdims=True)
        s = z - m
        lse = jnp.log(jnp.sum(jnp.exp(s), axis=0, keepdims=True))
        out_ref[b] = s - lse


def _build_acat(conv_w, conv_b, h_out, sb):
    n_ch = conv_w.shape[0]
    h_in = h_out + 2
    w = conv_w.reshape(n_ch, 3, 3).astype(jnp.float32)
    hh = jnp.arange(h_out)
    tt = jnp.arange(h_in)
    kk = jnp.arange(3)
    sel = (tt[None, :, None] == hh[:, None, None] + kk[None, None, :])
    sel = sel.astype(jnp.float32)
    cols = []
    for kj in range(3):
        blk = jnp.einsum('htk,ck->cht', sel, w[:, :, kj])
        blk = blk.reshape(h_out * n_ch, h_in)
        pad = jnp.zeros((h_out * n_ch, sb - h_in), jnp.float32)
        if kj == 0:
            bias_col = jnp.repeat(conv_b.astype(jnp.float32),
                                  h_out).reshape(h_out * n_ch, 1)
            pad = jnp.concatenate(
                [bias_col,
                 jnp.zeros((h_out * n_ch, sb - h_in - 1), jnp.float32)],
                axis=1)
        cols.append(jnp.concatenate([blk, pad], axis=1))
    return jnp.concatenate(cols, axis=1)


def kernel(x, conv_w, conv_b, fc_w, fc_b):
    n, c_in, h_in, w_in = x.shape
    assert c_in == 1
    h_out = h_in - 2
    w_out = w_in - 2
    n_ch = conv_w.shape[0]
    n_cls = fc_w.shape[0]
    chunk = 256
    assert w_out % chunk == 0
    n_chunks = w_out // chunk
    b_blk = 8 if n % 8 == 0 else (2 if n % 2 == 0 else 1)
    sb = ((h_in + 1 + 7) // 8) * 8
    n_rows = n_ch * h_out
    w_pad = ((w_in + 127) // 128) * 128

    acat = _build_acat(conv_w, conv_b, h_out, sb)
    wfct = fc_w.astype(jnp.float32).T
    bfc2 = fc_b.astype(jnp.float32).reshape(1, n_cls)

    body = functools.partial(
        _net_body, b_blk=b_blk, h_in=h_in, w_in=w_in, sb=sb,
        n_chunks=n_chunks, chunk=chunk, n_ch=n_ch, n_cls=n_cls)

    out = pl.pallas_call(
        body,
        out_shape=jax.ShapeDtypeStruct((n, n_ch, h_out, n_cls), jnp.float32),
        grid=(n // b_blk,),
        in_specs=[
            pl.BlockSpec((b_blk, 1, h_in, w_in), lambda i: (i, 0, 0, 0)),
            pl.BlockSpec((n_rows, 3 * sb), lambda i: (0, 0)),
            pl.BlockSpec((w_out, n_cls), lambda i: (0, 0)),
            pl.BlockSpec((1, n_cls), lambda i: (0, 0)),
        ],
        out_specs=pl.BlockSpec((b_blk, n_ch, h_out, n_cls),
                               lambda i: (i, 0, 0, 0)),
        scratch_shapes=[pltpu.VMEM((b_blk, 3 * sb, w_pad), jnp.float32)],
        compiler_params=pltpu.CompilerParams(
            dimension_semantics=("parallel",),
            vmem_limit_bytes=64 * 1024 * 1024),
    )(x, acat, wfct, bfc2)

    return out
